# Initial kernel scaffold; baseline (speedup 1.0000x reference)
import functools

import jax
import jax.numpy as jnp
from jax import lax
from jax.experimental import pallas as pl
from jax.experimental.pallas import tpu as pltpu
from jax.experimental.pallas import tpu_sc as plsc

VOCAB = 100000
N_SEG = 3
MAX_LEN = 200
EMB = 64
BATCH = 4096

NC = 2
NS = 16
NW = NC * NS
SEQ_PER_W = BATCH // NW
HALF = MAX_LEN // 2
NV = EMB // 16


def _body(seq_hbm, lbl_hbm, tok_hbm, seg_hbm, pos_hbm, pidx_hbm, out_hbm,
          posseg_v, pos_v, seg_v, idx_v, lbl_v, rows_v, pidx_v,
          gsem, ssem):
    wid = lax.axis_index("s") * NC + lax.axis_index("c")

    pltpu.sync_copy(pidx_hbm, pidx_v)
    pltpu.sync_copy(seg_hbm, seg_v)
    for h in range(2):
        pltpu.async_copy(pos_hbm.at[pidx_v.at[h]],
                         pos_v.at[pl.ds(h * HALF, HALF)], gsem).wait()
    seg_vals = [[seg_v[s, pl.ds(j * 16, 16)] for j in range(NV)]
                for s in range(N_SEG)]

    def init_body(l, _):
        for j in range(NV):
            p = pos_v[l, pl.ds(j * 16, 16)]
            for s in range(N_SEG):
                posseg_v[l * N_SEG + s, pl.ds(j * 16, 16)] = p + seg_vals[s][j]
        return 0

    lax.fori_loop(0, MAX_LEN, init_body, 0)

    def seq_body(i, _):
        b = wid * SEQ_PER_W + i
        pltpu.sync_copy(seq_hbm.at[b], idx_v)
        pltpu.sync_copy(lbl_hbm.at[b], lbl_v)
        for h in range(2):
            pltpu.async_copy(tok_hbm.at[idx_v.at[h]],
                             rows_v.at[pl.ds(h * HALF, HALF)], gsem).wait()

        def row_body(r, _):
            s = lbl_v[r // HALF, r % HALF]
            c = r * N_SEG + s
            for j in range(NV):
                rows_v[r, pl.ds(j * 16, 16)] = (
                    rows_v[r, pl.ds(j * 16, 16)]
                    + posseg_v[c, pl.ds(j * 16, 16)])
            return 0

        lax.fori_loop(0, MAX_LEN, row_body, 0)
        pltpu.async_copy(rows_v, out_hbm.at[b], ssem).wait()
        return 0

    lax.fori_loop(0, SEQ_PER_W, seq_body, 0)


def kernel(sequence, segment_label, token_table, segment_table,
           position_table, pos_inp):
    seq = jnp.asarray(sequence, jnp.int32).reshape(BATCH, 2, HALF)
    lbl = jnp.asarray(segment_label, jnp.int32).reshape(BATCH, 2, HALF)
    pidx = jnp.asarray(pos_inp, jnp.int32).reshape(2, HALF)

    run = pl.kernel(
        _body,
        out_type=jax.ShapeDtypeStruct((BATCH, MAX_LEN, EMB), jnp.float32),
        mesh=plsc.VectorSubcoreMesh(core_axis_name="c", subcore_axis_name="s"),
        scratch_types=[
            pltpu.VMEM((MAX_LEN * N_SEG, EMB), jnp.float32),
            pltpu.VMEM((MAX_LEN, EMB), jnp.float32),
            pltpu.VMEM((N_SEG, EMB), jnp.float32),
            pltpu.VMEM((2, HALF), jnp.int32),
            pltpu.VMEM((2, HALF), jnp.int32),
            pltpu.VMEM((MAX_LEN, EMB), jnp.float32),
            pltpu.VMEM((2, HALF), jnp.int32),
            pltpu.SemaphoreType.DMA,
            pltpu.SemaphoreType.DMA,
        ],
    )
    return run(seq, lbl, token_table, segment_table, position_table, pidx)


# speedup vs baseline: 3.8647x; 3.8647x over previous
import functools

import jax
import jax.numpy as jnp
from jax import lax
from jax.experimental import pallas as pl
from jax.experimental.pallas import tpu as pltpu
from jax.experimental.pallas import tpu_sc as plsc

VOCAB = 100000
N_SEG = 3
MAX_LEN = 200
EMB = 64
BATCH = 4096

NC = 2
NS = 16
NW = NC * NS
SEQ_PER_W = BATCH // NW
HALF = MAX_LEN // 2
NV = EMB // 16


def _body(seq_hbm, lbl_hbm, tok_hbm, seg_hbm, pos_hbm, pidx_hbm, out_hbm,
          posseg_v, pos_v, seg_v, idx_v, lbl_v, rows_v, pidx_v,
          gsem, ssem):
    wid = lax.axis_index("s") * NC + lax.axis_index("c")

    pltpu.sync_copy(pidx_hbm, pidx_v)
    pltpu.sync_copy(seg_hbm, seg_v)
    for h in range(2):
        pltpu.async_copy(pos_hbm.at[pidx_v.at[h]],
                         pos_v.at[pl.ds(h * HALF, HALF)], gsem).wait()
    seg_vals = [[seg_v[s, pl.ds(j * 16, 16)] for j in range(NV)]
                for s in range(N_SEG)]

    def init_body(l, _):
        for j in range(NV):
            p = pos_v[l, pl.ds(j * 16, 16)]
            for s in range(N_SEG):
                posseg_v[l * N_SEG + s, pl.ds(j * 16, 16)] = p + seg_vals[s][j]
        return 0

    lax.fori_loop(0, MAX_LEN, init_body, 0)

    def do_rows(r0, sv, lanes):
        pc = r0 * N_SEG
        for k in range(lanes):
            s = sv[k]
            c = pc + (k * N_SEG) + s
            for j in range(NV):
                rows_v[r0 + k, pl.ds(j * 16, 16)] = (
                    rows_v[r0 + k, pl.ds(j * 16, 16)]
                    + posseg_v[c, pl.ds(j * 16, 16)])

    def seq_body(i, _):
        b = wid * SEQ_PER_W + i
        pltpu.sync_copy(seq_hbm.at[b], idx_v)
        pltpu.sync_copy(lbl_hbm.at[b], lbl_v)
        for h in range(2):
            pltpu.async_copy(tok_hbm.at[idx_v.at[h]],
                             rows_v.at[pl.ds(h * HALF, HALF)], gsem).wait()

        def group_body(g, _):
            r0 = g * 16
            sv = lbl_v[pl.ds(r0, 16)]
            do_rows(r0, sv, 16)
            return 0

        lax.fori_loop(0, MAX_LEN // 16, group_body, 0)
        tail = MAX_LEN % 16
        if tail:
            svt = lbl_v[pl.ds(MAX_LEN - 16, 16)]
            pc = (MAX_LEN - tail) * N_SEG
            for k in range(tail):
                s = svt[16 - tail + k]
                c = pc + (k * N_SEG) + s
                r = MAX_LEN - tail + k
                for j in range(NV):
                    rows_v[r, pl.ds(j * 16, 16)] = (
                        rows_v[r, pl.ds(j * 16, 16)]
                        + posseg_v[c, pl.ds(j * 16, 16)])
        pltpu.async_copy(rows_v, out_hbm.at[b], ssem).wait()
        return 0

    lax.fori_loop(0, SEQ_PER_W, seq_body, 0)


def kernel(sequence, segment_label, token_table, segment_table,
           position_table, pos_inp):
    seq = jnp.asarray(sequence, jnp.int32).reshape(BATCH, 2, HALF)
    lbl = jnp.asarray(segment_label, jnp.int32).reshape(BATCH, MAX_LEN)
    pidx = jnp.asarray(pos_inp, jnp.int32).reshape(2, HALF)

    run = pl.kernel(
        _body,
        out_type=jax.ShapeDtypeStruct((BATCH, MAX_LEN, EMB), jnp.float32),
        mesh=plsc.VectorSubcoreMesh(core_axis_name="c", subcore_axis_name="s"),
        compiler_params=pltpu.CompilerParams(use_tc_tiling_on_sc=False),
        scratch_types=[
            pltpu.VMEM((MAX_LEN * N_SEG, EMB), jnp.float32),
            pltpu.VMEM((MAX_LEN, EMB), jnp.float32),
            pltpu.VMEM((N_SEG, EMB), jnp.float32),
            pltpu.VMEM((2, HALF), jnp.int32),
            pltpu.VMEM((MAX_LEN,), jnp.int32),
            pltpu.VMEM((MAX_LEN, EMB), jnp.float32),
            pltpu.VMEM((2, HALF), jnp.int32),
            pltpu.SemaphoreType.DMA,
            pltpu.SemaphoreType.DMA,
        ],
    )
    return run(seq, lbl, token_table, segment_table, position_table, pidx)


# speedup vs baseline: 5.2782x; 1.3657x over previous
import functools

import jax
import jax.numpy as jnp
from jax import lax
from jax.experimental import pallas as pl
from jax.experimental.pallas import tpu as pltpu
from jax.experimental.pallas import tpu_sc as plsc

VOCAB = 100000
N_SEG = 3
MAX_LEN = 200
EMB = 64
BATCH = 4096

NC = 2
NS = 16
NW = NC * NS
NSEQ = BATCH // NW
HALF = MAX_LEN // 2
NV = EMB // 16
SLOTS = 4


def _body(seq_hbm, lbl_hbm, tok_hbm, seg_hbm, pos_hbm, pidx_hbm, out_hbm,
          posseg_v, pos_v, seg_v, idx_v, lbl_v, rows_v, pidx_v,
          isems, gsems, ssems):
    wid = lax.axis_index("s") * NC + lax.axis_index("c")
    base = wid * NSEQ

    pltpu.sync_copy(pidx_hbm, pidx_v)
    pltpu.sync_copy(seg_hbm, seg_v)
    for h in range(2):
        pltpu.async_copy(pos_hbm.at[pidx_v.at[h]],
                         pos_v.at[pl.ds(h * HALF, HALF)], gsems[0]).wait()
    seg_vals = [[seg_v[s, pl.ds(j * 16, 16)] for j in range(NV)]
                for s in range(N_SEG)]

    def init_body(l, _):
        for j in range(NV):
            p = pos_v[l, pl.ds(j * 16, 16)]
            for s in range(N_SEG):
                posseg_v[l * N_SEG + s, pl.ds(j * 16, 16)] = p + seg_vals[s][j]
        return 0

    lax.fori_loop(0, MAX_LEN, init_body, 0)

    def start_idx(i, sl):
        pltpu.async_copy(seq_hbm.at[base + i], idx_v.at[sl], isems[sl])
        pltpu.async_copy(lbl_hbm.at[base + i], lbl_v.at[sl], isems[sl])

    def wait_idx(sl):
        pltpu.make_async_copy(seq_hbm.at[0], idx_v.at[sl], isems[sl]).wait()
        pltpu.make_async_copy(lbl_hbm.at[0], lbl_v.at[sl], isems[sl]).wait()

    def start_gather(sl):
        for h in range(2):
            pltpu.async_copy(tok_hbm.at[idx_v.at[sl].at[h]],
                             rows_v.at[sl].at[pl.ds(h * HALF, HALF)],
                             gsems[sl])

    def wait_gather(sl):
        for h in range(2):
            pltpu.make_async_copy(tok_hbm.at[idx_v.at[sl].at[h]],
                                  rows_v.at[sl].at[pl.ds(h * HALF, HALF)],
                                  gsems[sl]).wait()

    def start_scatter(i, sl):
        pltpu.async_copy(rows_v.at[sl], out_hbm.at[base + i], ssems[sl])

    def wait_scatter(sl):
        pltpu.make_async_copy(rows_v.at[sl], out_hbm.at[0], ssems[sl]).wait()

    def compute(sl):
        lblr = lbl_v.at[sl]
        rowr = rows_v.at[sl]

        def add_rows(r0, sv, lane0):
            pc = (r0 - lane0) * N_SEG
            for k in range(lane0, 16):
                s = sv[k]
                c = pc + (k * N_SEG) + s
                r = r0 + (k - lane0)
                for j in range(NV):
                    rowr[r, pl.ds(j * 16, 16)] = (
                        rowr[r, pl.ds(j * 16, 16)]
                        + posseg_v[c, pl.ds(j * 16, 16)])

        def group_body(g, _):
            r0 = g * 16
            add_rows(r0, lblr[pl.ds(r0, 16)], 0)
            return 0

        lax.fori_loop(0, MAX_LEN // 16, group_body, 0)
        add_rows(MAX_LEN - 8, lblr[pl.ds(MAX_LEN - 16, 16)], 8)

    start_idx(0, 0)
    start_idx(1, 1)
    start_idx(2, 2)
    wait_idx(0)
    start_gather(0)
    wait_idx(1)
    start_gather(1)

    def macro_body(m, _):
        i0 = m * SLOTS
        for u in range(SLOTS):
            i = i0 + u
            sl = u
            sl2 = (u + 2) % SLOTS
            sl3 = (u + 3) % SLOTS

            @pl.when(i + 3 < NSEQ)
            def _():
                start_idx(i + 3, sl3)

            @pl.when(i + 2 < NSEQ)
            def _():
                @pl.when(i >= 2)
                def _():
                    wait_scatter(sl2)
                wait_idx(sl2)
                start_gather(sl2)

            wait_gather(sl)
            compute(sl)
            start_scatter(i, sl)
        return 0

    lax.fori_loop(0, NSEQ // SLOTS, macro_body, 0)
    for sl in range(SLOTS):
        wait_scatter(sl)


def kernel(sequence, segment_label, token_table, segment_table,
           position_table, pos_inp):
    seq = jnp.asarray(sequence, jnp.int32).reshape(BATCH, 2, HALF)
    lbl = jnp.asarray(segment_label, jnp.int32).reshape(BATCH, MAX_LEN)
    pidx = jnp.asarray(pos_inp, jnp.int32).reshape(2, HALF)

    run = pl.kernel(
        _body,
        out_type=jax.ShapeDtypeStruct((BATCH, MAX_LEN, EMB), jnp.float32),
        mesh=plsc.VectorSubcoreMesh(core_axis_name="c", subcore_axis_name="s"),
        compiler_params=pltpu.CompilerParams(use_tc_tiling_on_sc=False),
        scratch_types=[
            pltpu.VMEM((MAX_LEN * N_SEG, EMB), jnp.float32),
            pltpu.VMEM((MAX_LEN, EMB), jnp.float32),
            pltpu.VMEM((N_SEG, EMB), jnp.float32),
            pltpu.VMEM((SLOTS, 2, HALF), jnp.int32),
            pltpu.VMEM((SLOTS, MAX_LEN), jnp.int32),
            pltpu.VMEM((SLOTS, MAX_LEN, EMB), jnp.float32),
            pltpu.VMEM((2, HALF), jnp.int32),
            [pltpu.SemaphoreType.DMA] * SLOTS,
            [pltpu.SemaphoreType.DMA] * SLOTS,
            [pltpu.SemaphoreType.DMA] * SLOTS,
        ],
    )
    return run(seq, lbl, token_table, segment_table, position_table, pidx)


# speedup vs baseline: 7.8281x; 1.4831x over previous
import functools

import jax
import jax.numpy as jnp
from jax import lax
from jax.experimental import pallas as pl
from jax.experimental.pallas import tpu as pltpu
from jax.experimental.pallas import tpu_sc as plsc

VOCAB = 100000
N_SEG = 3
MAX_LEN = 200
EMB = 64
BATCH = 4096

NC = 2
NS = 16
NW = NC * NS
NSEQ = BATCH // NW
HALF = MAX_LEN // 2
NV = EMB // 16
SLOTS = 4
OFFS = (0, 16, 32, 48, 64, 80, 84)


def _body(seq_hbm, lbl_hbm, tok_hbm, seg_hbm, pos_hbm, pidx_hbm, out_hbm,
          pos_v, seg_v, pidx_v, idx_v, lbl_v, cidx_v, rows_v, posseg_sh,
          bsem, isems, psems, gsems, ssems):
    sid = lax.axis_index("s")
    wid = sid * NC + lax.axis_index("c")
    base = wid * NSEQ

    @pl.when(sid == 0)
    def _():
        pltpu.sync_copy(pidx_hbm, pidx_v)
        pltpu.sync_copy(seg_hbm, seg_v)
        for h in range(2):
            pltpu.async_copy(pos_hbm.at[pidx_v.at[h]],
                             pos_v.at[pl.ds(h * HALF, HALF)], bsem).wait()
        for s in range(N_SEG):
            seg_vals = [seg_v[s, pl.ds(j * 16, 16)] for j in range(NV)]

            def seg_body(l, _, s=s, seg_vals=seg_vals):
                for j in range(NV):
                    rows_v[s, l, pl.ds(j * 16, 16)] = (
                        pos_v[l, pl.ds(j * 16, 16)] + seg_vals[j])
                return 0

            lax.fori_loop(0, MAX_LEN, seg_body, 0)
            pltpu.sync_copy(rows_v.at[s],
                            posseg_sh.at[pl.ds(s * MAX_LEN, MAX_LEN)])
    plsc.subcore_barrier()

    def start_idx(i, sl):
        pltpu.async_copy(seq_hbm.at[base + i], idx_v.at[sl], isems[sl])
        pltpu.async_copy(lbl_hbm.at[base + i], lbl_v.at[sl], isems[sl])

    def wait_idx(sl):
        pltpu.make_async_copy(seq_hbm.at[0], idx_v.at[sl], isems[sl]).wait()
        pltpu.make_async_copy(lbl_hbm.at[0], lbl_v.at[sl], isems[sl]).wait()

    def cidx_compute(sl):
        for h in range(2):
            for off in OFFS:
                r0 = h * HALF + off
                lv = lax.iota(jnp.int32, 16) + r0
                sv = lbl_v[sl, pl.ds(r0, 16)]
                cidx_v[sl, h, pl.ds(off, 16)] = sv * MAX_LEN + lv

    def start_posseg(sl):
        for h in range(2):
            pltpu.async_copy(posseg_sh.at[cidx_v.at[sl].at[h]],
                             rows_v.at[sl].at[pl.ds(h * HALF, HALF)],
                             psems[sl])

    def wait_posseg(sl):
        for h in range(2):
            pltpu.make_async_copy(posseg_sh.at[cidx_v.at[sl].at[h]],
                                  rows_v.at[sl].at[pl.ds(h * HALF, HALF)],
                                  psems[sl]).wait()

    def start_tokadd(sl):
        for h in range(2):
            pltpu.async_copy(tok_hbm.at[idx_v.at[sl].at[h]],
                             rows_v.at[sl].at[pl.ds(h * HALF, HALF)],
                             gsems[sl], add=True)

    def wait_tokadd(sl):
        for h in range(2):
            pltpu.make_async_copy(tok_hbm.at[idx_v.at[sl].at[h]],
                                  rows_v.at[sl].at[pl.ds(h * HALF, HALF)],
                                  gsems[sl]).wait()

    def start_scatter(i, sl):
        pltpu.async_copy(rows_v.at[sl], out_hbm.at[base + i], ssems[sl])

    def wait_scatter(sl):
        pltpu.make_async_copy(rows_v.at[sl], out_hbm.at[0], ssems[sl]).wait()

    start_idx(0, 0)
    start_idx(1, 1)
    start_idx(2, 2)
    wait_idx(0)
    cidx_compute(0)
    start_posseg(0)
    wait_idx(1)
    cidx_compute(1)
    start_posseg(1)
    wait_posseg(0)
    start_tokadd(0)

    def macro_body(m, _):
        i0 = m * SLOTS
        for u in range(SLOTS):
            i = i0 + u
            sl = u
            sl1 = (u + 1) % SLOTS
            sl2 = (u + 2) % SLOTS
            sl3 = (u + 3) % SLOTS

            @pl.when(i + 3 < NSEQ)
            def _():
                start_idx(i + 3, sl3)

            @pl.when(i + 2 < NSEQ)
            def _():
                @pl.when(i >= 2)
                def _():
                    wait_scatter(sl2)
                wait_idx(sl2)
                cidx_compute(sl2)
                start_posseg(sl2)

            @pl.when(i + 1 < NSEQ)
            def _():
                wait_posseg(sl1)
                start_tokadd(sl1)

            wait_tokadd(sl)
            start_scatter(i, sl)
        return 0

    lax.fori_loop(0, NSEQ // SLOTS, macro_body, 0)
    for sl in range(SLOTS):
        wait_scatter(sl)


def kernel(sequence, segment_label, token_table, segment_table,
           position_table, pos_inp):
    seq = jnp.asarray(sequence, jnp.int32).reshape(BATCH, 2, HALF)
    lbl = jnp.asarray(segment_label, jnp.int32).reshape(BATCH, MAX_LEN)
    pidx = jnp.asarray(pos_inp, jnp.int32).reshape(2, HALF)

    run = pl.kernel(
        _body,
        out_type=jax.ShapeDtypeStruct((BATCH, MAX_LEN, EMB), jnp.float32),
        mesh=plsc.VectorSubcoreMesh(core_axis_name="c", subcore_axis_name="s"),
        compiler_params=pltpu.CompilerParams(use_tc_tiling_on_sc=False),
        scratch_types=[
            pltpu.VMEM((MAX_LEN, EMB), jnp.float32),
            pltpu.VMEM((N_SEG, EMB), jnp.float32),
            pltpu.VMEM((2, HALF), jnp.int32),
            pltpu.VMEM((SLOTS, 2, HALF), jnp.int32),
            pltpu.VMEM((SLOTS, MAX_LEN), jnp.int32),
            pltpu.VMEM((SLOTS, 2, HALF), jnp.int32),
            pltpu.VMEM((SLOTS, MAX_LEN, EMB), jnp.float32),
            pltpu.VMEM_SHARED((N_SEG * MAX_LEN, EMB), jnp.float32),
            pltpu.SemaphoreType.DMA,
            [pltpu.SemaphoreType.DMA] * SLOTS,
            [pltpu.SemaphoreType.DMA] * SLOTS,
            [pltpu.SemaphoreType.DMA] * SLOTS,
            [pltpu.SemaphoreType.DMA] * SLOTS,
        ],
    )
    return run(seq, lbl, token_table, segment_table, position_table, pidx)


# speedup vs baseline: 13.3611x; 1.7068x over previous
import functools

import jax
import jax.numpy as jnp
from jax import lax
from jax.experimental import pallas as pl
from jax.experimental.pallas import tpu as pltpu
from jax.experimental.pallas import tpu_sc as plsc

VOCAB = 100000
N_SEG = 3
MAX_LEN = 200
EMB = 64
BATCH = 4096

NC = 2
NS = 16
NW = NC * NS
NSEQ = BATCH // NW
HALF = MAX_LEN // 2
NV = EMB // 16
SLOTS = 4
OFFS = (0, 16, 32, 48, 64, 80, 84)


def _body(seq_hbm, lbl_hbm, tok_hbm, seg_hbm, pos_hbm, pidx_hbm, out_hbm,
          pos_v, seg_v, pidx_v, idx_v, lbl_v, cidx_v, rows_v, posseg_sh,
          bsem, isems, psems, gsems, ssems):
    sid = lax.axis_index("s")
    wid = sid * NC + lax.axis_index("c")
    base = wid * NSEQ

    @pl.when(sid == 0)
    def _():
        pltpu.sync_copy(pidx_hbm, pidx_v)
        pltpu.sync_copy(seg_hbm, seg_v)
        for h in range(2):
            pltpu.async_copy(pos_hbm.at[pidx_v.at[h]],
                             pos_v.at[pl.ds(h * HALF, HALF)], bsem).wait()
        for s in range(N_SEG):
            seg_vals = [seg_v[s, pl.ds(j * 16, 16)] for j in range(NV)]

            def seg_body(l, _, s=s, seg_vals=seg_vals):
                for j in range(NV):
                    rows_v[s, l, pl.ds(j * 16, 16)] = (
                        pos_v[l, pl.ds(j * 16, 16)] + seg_vals[j])
                return 0

            lax.fori_loop(0, MAX_LEN, seg_body, 0)
            pltpu.sync_copy(rows_v.at[s],
                            posseg_sh.at[pl.ds(s * MAX_LEN, MAX_LEN)])
    plsc.subcore_barrier()

    def start_idx(i, sl):
        pltpu.async_copy(seq_hbm.at[base + i], idx_v.at[sl], isems[sl])
        pltpu.async_copy(lbl_hbm.at[base + i], lbl_v.at[sl], isems[sl])

    def wait_idx(sl):
        pltpu.make_async_copy(seq_hbm.at[0], idx_v.at[sl], isems[sl]).wait()
        pltpu.make_async_copy(lbl_hbm.at[0], lbl_v.at[sl], isems[sl]).wait()

    def cidx_compute(sl):
        for h in range(2):
            for off in OFFS:
                r0 = h * HALF + off
                lv = lax.iota(jnp.int32, 16) + r0
                sv = lbl_v[sl, pl.ds(r0, 16)]
                cidx_v[sl, h, pl.ds(off, 16)] = sv * MAX_LEN + lv

    def start_posseg(sl):
        for h in range(2):
            pltpu.async_copy(posseg_sh.at[cidx_v.at[sl].at[h]],
                             rows_v.at[sl].at[pl.ds(h * HALF, HALF)],
                             psems[sl])

    def wait_posseg(sl):
        for h in range(2):
            pltpu.make_async_copy(posseg_sh.at[cidx_v.at[sl].at[h]],
                                  rows_v.at[sl].at[pl.ds(h * HALF, HALF)],
                                  psems[sl]).wait()

    def start_tokadd(sl):
        for h in range(2):
            pltpu.async_copy(tok_hbm.at[idx_v.at[sl].at[h]],
                             rows_v.at[sl].at[pl.ds(h * HALF, HALF)],
                             gsems[sl], add=True)

    def wait_tokadd(sl):
        for h in range(2):
            pltpu.make_async_copy(tok_hbm.at[idx_v.at[sl].at[h]],
                                  rows_v.at[sl].at[pl.ds(h * HALF, HALF)],
                                  gsems[sl]).wait()

    def start_scatter(i, sl):
        pltpu.async_copy(rows_v.at[sl],
                         out_hbm.at[base + i].at[:, pl.ds(0, EMB)], ssems[sl])

    def wait_scatter(sl):
        pltpu.make_async_copy(rows_v.at[sl],
                              out_hbm.at[0].at[:, pl.ds(0, EMB)],
                              ssems[sl]).wait()

    start_idx(0, 0)
    start_idx(1, 1)
    start_idx(2, 2)
    wait_idx(0)
    cidx_compute(0)
    start_posseg(0)
    wait_idx(1)
    cidx_compute(1)
    start_posseg(1)
    wait_posseg(0)
    start_tokadd(0)

    def macro_body(m, _):
        i0 = m * SLOTS
        for u in range(SLOTS):
            i = i0 + u
            sl = u
            sl1 = (u + 1) % SLOTS
            sl2 = (u + 2) % SLOTS
            sl3 = (u + 3) % SLOTS

            @pl.when(i + 3 < NSEQ)
            def _():
                start_idx(i + 3, sl3)

            @pl.when(i + 2 < NSEQ)
            def _():
                @pl.when(i >= 2)
                def _():
                    wait_scatter(sl2)
                wait_idx(sl2)
                cidx_compute(sl2)
                start_posseg(sl2)

            @pl.when(i + 1 < NSEQ)
            def _():
                wait_posseg(sl1)
                start_tokadd(sl1)

            wait_tokadd(sl)
            start_scatter(i, sl)
        return 0

    lax.fori_loop(0, NSEQ // SLOTS, macro_body, 0)
    for sl in range(SLOTS):
        wait_scatter(sl)


def kernel(sequence, segment_label, token_table, segment_table,
           position_table, pos_inp):
    seq = jnp.asarray(sequence, jnp.int32).reshape(BATCH, 2, HALF)
    lbl = jnp.asarray(segment_label, jnp.int32).reshape(BATCH, MAX_LEN)
    pidx = jnp.asarray(pos_inp, jnp.int32).reshape(2, HALF)

    run = pl.kernel(
        _body,
        out_type=jax.ShapeDtypeStruct((BATCH, MAX_LEN, 2 * EMB), jnp.float32),
        mesh=plsc.VectorSubcoreMesh(core_axis_name="c", subcore_axis_name="s"),
        compiler_params=pltpu.CompilerParams(use_tc_tiling_on_sc=False),
        scratch_types=[
            pltpu.VMEM((MAX_LEN, EMB), jnp.float32),
            pltpu.VMEM((N_SEG, EMB), jnp.float32),
            pltpu.VMEM((2, HALF), jnp.int32),
            pltpu.VMEM((SLOTS, 2, HALF), jnp.int32),
            pltpu.VMEM((SLOTS, MAX_LEN), jnp.int32),
            pltpu.VMEM((SLOTS, 2, HALF), jnp.int32),
            pltpu.VMEM((SLOTS, MAX_LEN, EMB), jnp.float32),
            pltpu.VMEM_SHARED((N_SEG * MAX_LEN, EMB), jnp.float32),
            pltpu.SemaphoreType.DMA,
            [pltpu.SemaphoreType.DMA] * SLOTS,
            [pltpu.SemaphoreType.DMA] * SLOTS,
            [pltpu.SemaphoreType.DMA] * SLOTS,
            [pltpu.SemaphoreType.DMA] * SLOTS,
        ],
    )
    out = run(seq, lbl, token_table, segment_table, position_table, pidx)
    return out[:, :, :EMB]
